# baseline (speedup 1.0000x reference)
import functools

import jax
import jax.numpy as jnp
from jax import lax
from jax.experimental import pallas as pl
from jax.experimental.pallas import tpu as pltpu
from jax.experimental.pallas import tpu_sc as plsc

N = 10000
E = 320000
D = 128
EPS = 1e-5

NC = 2
NS = 16
NW = NC * NS
CHUNK = 128
NBUF = 2
IGRP = 40
C = 80
NGRP = C // IGRP
E_PAD = NW * C * CHUNK
ZPAD = 128
NROWS = 10240
RPS = NROWS // NS
NA = 10112
ARPS = NA // NS

_mesh = plsc.VectorSubcoreMesh(core_axis_name="c", subcore_axis_name="s")


@functools.partial(
    pl.kernel,
    out_type=jax.ShapeDtypeStruct((NC, NROWS), jnp.float32),
    mesh=_mesh,
    scratch_types=[
        pltpu.VMEM((C, CHUNK), jnp.int32),
        pltpu.VMEM((CHUNK,), jnp.float32),
        pltpu.VMEM_SHARED((NROWS,), jnp.float32),
    ],
)
def _deg_kernel(dst_hbm, zeros1_hbm, out_hbm, dst_v, ones_v, acc_sh):
    c = lax.axis_index("c")
    s = lax.axis_index("s")
    w = s * NC + c

    pltpu.sync_copy(dst_hbm.at[w], dst_v)
    for i in range(CHUNK // 16):
        ones_v[pl.ds(i * 16, 16)] = jnp.ones((16,), jnp.float32)

    pltpu.sync_copy(zeros1_hbm, acc_sh.at[pl.ds(s * RPS, RPS)])
    plsc.subcore_barrier()

    @pl.loop(0, C)
    def _(j):
        pltpu.sync_copy(ones_v, acc_sh.at[dst_v.at[j]], add=True)

    plsc.subcore_barrier()
    pltpu.sync_copy(acc_sh.at[pl.ds(s * RPS, RPS)],
                    out_hbm.at[c, pl.ds(s * RPS, RPS)])


@functools.partial(
    pl.kernel,
    out_type=jax.ShapeDtypeStruct((NC, NA, D), jnp.float32),
    mesh=_mesh,
    scratch_types=[
        pltpu.VMEM((IGRP, CHUNK), jnp.int32),
        pltpu.VMEM((IGRP, CHUNK), jnp.int32),
        pltpu.VMEM((NBUF, CHUNK, D), jnp.float32),
        pltpu.VMEM_SHARED((NA, D), jnp.float32),
        [pltpu.SemaphoreType.DMA] * NBUF,
        [pltpu.SemaphoreType.DMA] * NBUF,
    ],
)
def _agg_kernel(g_hbm, src_hbm, dst_hbm, zeros2_hbm, out_hbm,
                src_v, dst_v, rows_v, acc_sh, gsems, ssems):
    c = lax.axis_index("c")
    s = lax.axis_index("s")
    w = s * NC + c

    pltpu.sync_copy(zeros2_hbm, rows_v.at[0])
    for i in range(ARPS // CHUNK):
        pltpu.async_copy(rows_v.at[0],
                         acc_sh.at[pl.ds(s * ARPS + i * CHUNK, CHUNK)],
                         gsems[0])
    rem = ARPS % CHUNK
    pltpu.sync_copy(zeros2_hbm.at[pl.ds(0, rem)],
                    acc_sh.at[pl.ds(s * ARPS + (ARPS // CHUNK) * CHUNK,
                                    rem)])
    for i in range(ARPS // CHUNK):
        pltpu.make_async_copy(rows_v.at[0],
                              acc_sh.at[pl.ds(s * ARPS + i * CHUNK,
                                              CHUNK)],
                              gsems[0]).wait()
    plsc.subcore_barrier()

    def start_gather(b, j):
        pltpu.async_copy(g_hbm.at[src_v.at[j]], rows_v.at[b], gsems[b])

    def wait_gather(b, j):
        pltpu.make_async_copy(g_hbm.at[src_v.at[j]], rows_v.at[b],
                              gsems[b]).wait()

    def start_scatter(b, j):
        pltpu.async_copy(rows_v.at[b], acc_sh.at[dst_v.at[j]], ssems[b],
                         add=True)

    def wait_scatter(b, j):
        pltpu.make_async_copy(rows_v.at[b], acc_sh.at[dst_v.at[j]],
                              ssems[b]).wait()

    for gi in range(NGRP):
        pltpu.sync_copy(src_hbm.at[w, pl.ds(gi * IGRP, IGRP)], src_v)
        pltpu.sync_copy(dst_hbm.at[w, pl.ds(gi * IGRP, IGRP)], dst_v)

        for b in range(NBUF):
            start_gather(b, b)

        @pl.loop(0, IGRP // NBUF - 1)
        def _(t):
            for b in range(NBUF):
                wait_gather(b, t * NBUF + b)
                start_scatter(b, t * NBUF + b)
            for b in range(NBUF):
                wait_scatter(b, t * NBUF + b)
                start_gather(b, t * NBUF + b + NBUF)

        for b in range(NBUF):
            wait_gather(b, IGRP - NBUF + b)
            start_scatter(b, IGRP - NBUF + b)
        for b in range(NBUF):
            wait_scatter(b, IGRP - NBUF + b)

    plsc.subcore_barrier()
    pltpu.sync_copy(acc_sh.at[pl.ds(s * ARPS, ARPS)],
                    out_hbm.at[c, pl.ds(s * ARPS, ARPS)])


def _tc0_body(x_ref, w_ref, h_ref):
    h_ref[...] = jnp.dot(x_ref[...], w_ref[...],
                         preferred_element_type=jnp.float32)


def _tc1_body(h_ref, degp_ref, g_ref, dinv_ref):
    deg = degp_ref[0, :N] + degp_ref[1, :N] + 1.0
    dinv = lax.rsqrt(deg)[:, None]
    g_ref[:N] = h_ref[...] * dinv
    g_ref[N:] = jnp.zeros((ZPAD, D), jnp.float32)
    dinv_ref[...] = dinv


def _tc2_body(s_ref, g_ref, dinv_ref, b_ref, gam_ref, bet_ref, w_ref,
              out_ref):
    agg = s_ref[0, :N] + s_ref[1, :N] + g_ref[:N]
    t = agg * dinv_ref[...] + b_ref[...]
    t = jnp.maximum(t, 0.0)
    mean = jnp.mean(t, axis=0, keepdims=True)
    var = jnp.mean(jnp.square(t - mean), axis=0, keepdims=True)
    bn = (t - mean) * lax.rsqrt(var + EPS) * gam_ref[...] + bet_ref[...]
    h2 = jnp.dot(bn, w_ref[...], preferred_element_type=jnp.float32)
    out_ref[:N] = h2 * dinv_ref[...]
    out_ref[N:] = jnp.zeros((ZPAD, D), jnp.float32)


def _tc3_body(s_ref, g_ref, dinv_ref, b_ref, gam_ref, bet_ref, wc_ref,
              bc_ref, out_ref):
    agg = s_ref[0, :N] + s_ref[1, :N] + g_ref[:N]
    t = agg * dinv_ref[...] + b_ref[...]
    t = jnp.maximum(t, 0.0)
    mean = jnp.mean(t, axis=0, keepdims=True)
    var = jnp.mean(jnp.square(t - mean), axis=0, keepdims=True)
    bn = (t - mean) * lax.rsqrt(var + EPS) * gam_ref[...] + bet_ref[...]
    out_ref[...] = (
        jnp.dot(bn, wc_ref[...], preferred_element_type=jnp.float32)
        + bc_ref[...])


_tc0 = pl.pallas_call(
    _tc0_body,
    out_shape=jax.ShapeDtypeStruct((N, D), jnp.float32),
)

_tc1 = pl.pallas_call(
    _tc1_body,
    out_shape=(jax.ShapeDtypeStruct((N + ZPAD, D), jnp.float32),
               jax.ShapeDtypeStruct((N, 1), jnp.float32)),
)

_tc2 = pl.pallas_call(
    _tc2_body,
    out_shape=jax.ShapeDtypeStruct((N + ZPAD, D), jnp.float32),
)

_tc3 = pl.pallas_call(
    _tc3_body,
    out_shape=jax.ShapeDtypeStruct((N, 1), jnp.float32),
)


def kernel(x, edge_index, W1, b1, g1, be1, W2, b2, g2, be2, Wc, bc):
    pad = E_PAD - E
    ar = jnp.arange(pad, dtype=jnp.int32)
    src = jnp.concatenate([edge_index[0].astype(jnp.int32),
                           N + (ar % ZPAD)])
    dst_agg = jnp.concatenate([edge_index[1].astype(jnp.int32), ar % N])
    dst_deg = jnp.concatenate([edge_index[1].astype(jnp.int32),
                               N + (ar % CHUNK)])
    src_r = src.reshape(NW, C, CHUNK)
    dsta_r = dst_agg.reshape(NW, C, CHUNK)
    dstd_r = dst_deg.reshape(NW, C, CHUNK)

    zeros1 = jnp.zeros((RPS,), jnp.float32)
    zeros2 = jnp.zeros((CHUNK, D), jnp.float32)

    degp = _deg_kernel(dstd_r, zeros1)
    h1 = _tc0(x, W1)
    g1x, dinv = _tc1(h1, degp)

    s1 = _agg_kernel(g1x, src_r, dsta_r, zeros2)
    g2x = _tc2(s1, g1x, dinv, b1.reshape(1, D), g1.reshape(1, D),
               be1.reshape(1, D), W2)

    s2 = _agg_kernel(g2x, src_r, dsta_r, zeros2)
    out = _tc3(s2, g2x, dinv, b2.reshape(1, D), g2.reshape(1, D),
               be2.reshape(1, D), Wc, bc.reshape(1, 1))
    return out


# speedup vs baseline: 1.2405x; 1.2405x over previous
import functools

import jax
import jax.numpy as jnp
from jax import lax
from jax.experimental import pallas as pl
from jax.experimental.pallas import tpu as pltpu
from jax.experimental.pallas import tpu_sc as plsc

N = 10000
E = 320000
D = 128
EPS = 1e-5

NC = 2
NS = 16
NW = NC * NS
CHUNK = 128
NBUF = 2
IGRP = 40
C = 80
NGRP = C // IGRP
E_PAD = NW * C * CHUNK
ZPAD = 128
NROWS = 10240
RPS = NROWS // NS
NA = 10112
ARPS = NA // NS

_mesh = plsc.VectorSubcoreMesh(core_axis_name="c", subcore_axis_name="s")


@functools.partial(
    pl.kernel,
    out_type=jax.ShapeDtypeStruct((NC, NROWS), jnp.float32),
    mesh=_mesh,
    scratch_types=[
        pltpu.VMEM((C, CHUNK), jnp.int32),
        pltpu.VMEM((CHUNK,), jnp.float32),
        pltpu.VMEM_SHARED((NROWS,), jnp.float32),
    ],
)
def _deg_kernel(dst_hbm, zeros1_hbm, out_hbm, dst_v, ones_v, acc_sh):
    c = lax.axis_index("c")
    s = lax.axis_index("s")
    w = s * NC + c

    pltpu.sync_copy(dst_hbm.at[w], dst_v)
    for i in range(CHUNK // 16):
        ones_v[pl.ds(i * 16, 16)] = jnp.ones((16,), jnp.float32)

    pltpu.sync_copy(zeros1_hbm, acc_sh.at[pl.ds(s * RPS, RPS)])
    plsc.subcore_barrier()

    @pl.loop(0, C)
    def _(j):
        pltpu.sync_copy(ones_v, acc_sh.at[dst_v.at[j]], add=True)

    plsc.subcore_barrier()
    pltpu.sync_copy(acc_sh.at[pl.ds(s * RPS, RPS)],
                    out_hbm.at[c, pl.ds(s * RPS, RPS)])


@functools.partial(
    pl.kernel,
    out_type=jax.ShapeDtypeStruct((NC, NA, D), jnp.float32),
    mesh=_mesh,
    scratch_types=[
        pltpu.VMEM((IGRP, CHUNK), jnp.int32),
        pltpu.VMEM((IGRP, CHUNK), jnp.int32),
        pltpu.VMEM((NBUF, CHUNK, D), jnp.float32),
        pltpu.VMEM_SHARED((NA, D), jnp.float32),
        [pltpu.SemaphoreType.DMA] * NBUF,
        [pltpu.SemaphoreType.DMA] * NBUF,
    ],
)
def _agg_kernel(g_hbm, src_hbm, dst_hbm, zeros2_hbm, out_hbm,
                src_v, dst_v, rows_v, acc_sh, gsems, ssems):
    c = lax.axis_index("c")
    s = lax.axis_index("s")
    w = s * NC + c

    pltpu.sync_copy(zeros2_hbm, rows_v.at[0])
    for i in range(ARPS // CHUNK):
        pltpu.async_copy(rows_v.at[0],
                         acc_sh.at[pl.ds(s * ARPS + i * CHUNK, CHUNK)],
                         gsems[0])
    rem = ARPS % CHUNK
    pltpu.sync_copy(zeros2_hbm.at[pl.ds(0, rem)],
                    acc_sh.at[pl.ds(s * ARPS + (ARPS // CHUNK) * CHUNK,
                                    rem)])
    for i in range(ARPS // CHUNK):
        pltpu.make_async_copy(rows_v.at[0],
                              acc_sh.at[pl.ds(s * ARPS + i * CHUNK,
                                              CHUNK)],
                              gsems[0]).wait()
    plsc.subcore_barrier()

    H = CHUNK // 2

    def start_gather(b, j):
        pltpu.async_copy(g_hbm.at[src_v.at[j, pl.ds(0, H)]],
                         rows_v.at[b, pl.ds(0, H)], gsems[b])
        pltpu.async_copy(g_hbm.at[src_v.at[j, pl.ds(H, H)]],
                         rows_v.at[b, pl.ds(H, H)], gsems[b])

    def wait_gather(b, j):
        pltpu.make_async_copy(g_hbm.at[src_v.at[j, pl.ds(0, H)]],
                              rows_v.at[b, pl.ds(0, H)], gsems[b]).wait()
        pltpu.make_async_copy(g_hbm.at[src_v.at[j, pl.ds(H, H)]],
                              rows_v.at[b, pl.ds(H, H)], gsems[b]).wait()

    def start_scatter(b, j):
        pltpu.async_copy(rows_v.at[b], acc_sh.at[dst_v.at[j]], ssems[b],
                         add=True)

    def wait_scatter(b, j):
        pltpu.make_async_copy(rows_v.at[b], acc_sh.at[dst_v.at[j]],
                              ssems[b]).wait()

    for gi in range(NGRP):
        pltpu.sync_copy(src_hbm.at[w, pl.ds(gi * IGRP, IGRP)], src_v)
        pltpu.sync_copy(dst_hbm.at[w, pl.ds(gi * IGRP, IGRP)], dst_v)

        for b in range(NBUF):
            start_gather(b, b)

        @pl.loop(0, IGRP // NBUF - 1)
        def _(t):
            for b in range(NBUF):
                j = t * NBUF + b
                wait_gather(b, j)
                start_scatter(b, j)
                wait_scatter(b, j)
                start_gather(b, j + NBUF)

        for b in range(NBUF):
            j = IGRP - NBUF + b
            wait_gather(b, j)
            start_scatter(b, j)
            wait_scatter(b, j)

    plsc.subcore_barrier()
    pltpu.sync_copy(acc_sh.at[pl.ds(s * ARPS, ARPS)],
                    out_hbm.at[c, pl.ds(s * ARPS, ARPS)])


def _tc0_body(x_ref, w_ref, h_ref):
    h_ref[...] = jnp.dot(x_ref[...], w_ref[...],
                         preferred_element_type=jnp.float32)


def _tc1_body(h_ref, degp_ref, g_ref, dinv_ref):
    deg = degp_ref[0, :N] + degp_ref[1, :N] + 1.0
    dinv = lax.rsqrt(deg)[:, None]
    g_ref[:N] = h_ref[...] * dinv
    g_ref[N:] = jnp.zeros((ZPAD, D), jnp.float32)
    dinv_ref[...] = dinv


def _tc2_body(s_ref, g_ref, dinv_ref, b_ref, gam_ref, bet_ref, w_ref,
              out_ref):
    agg = s_ref[0, :N] + s_ref[1, :N] + g_ref[:N]
    t = agg * dinv_ref[...] + b_ref[...]
    t = jnp.maximum(t, 0.0)
    mean = jnp.mean(t, axis=0, keepdims=True)
    var = jnp.mean(jnp.square(t - mean), axis=0, keepdims=True)
    bn = (t - mean) * lax.rsqrt(var + EPS) * gam_ref[...] + bet_ref[...]
    h2 = jnp.dot(bn, w_ref[...], preferred_element_type=jnp.float32)
    out_ref[:N] = h2 * dinv_ref[...]
    out_ref[N:] = jnp.zeros((ZPAD, D), jnp.float32)


def _tc3_body(s_ref, g_ref, dinv_ref, b_ref, gam_ref, bet_ref, wc_ref,
              bc_ref, out_ref):
    agg = s_ref[0, :N] + s_ref[1, :N] + g_ref[:N]
    t = agg * dinv_ref[...] + b_ref[...]
    t = jnp.maximum(t, 0.0)
    mean = jnp.mean(t, axis=0, keepdims=True)
    var = jnp.mean(jnp.square(t - mean), axis=0, keepdims=True)
    bn = (t - mean) * lax.rsqrt(var + EPS) * gam_ref[...] + bet_ref[...]
    out_ref[...] = (
        jnp.dot(bn, wc_ref[...], preferred_element_type=jnp.float32)
        + bc_ref[...])


_tc0 = pl.pallas_call(
    _tc0_body,
    out_shape=jax.ShapeDtypeStruct((N, D), jnp.float32),
)

_tc1 = pl.pallas_call(
    _tc1_body,
    out_shape=(jax.ShapeDtypeStruct((N + ZPAD, D), jnp.float32),
               jax.ShapeDtypeStruct((N, 1), jnp.float32)),
)

_tc2 = pl.pallas_call(
    _tc2_body,
    out_shape=jax.ShapeDtypeStruct((N + ZPAD, D), jnp.float32),
)

_tc3 = pl.pallas_call(
    _tc3_body,
    out_shape=jax.ShapeDtypeStruct((N, 1), jnp.float32),
)


def kernel(x, edge_index, W1, b1, g1, be1, W2, b2, g2, be2, Wc, bc):
    pad = E_PAD - E
    ar = jnp.arange(pad, dtype=jnp.int32)
    src = jnp.concatenate([edge_index[0].astype(jnp.int32),
                           N + (ar % ZPAD)])
    dst_agg = jnp.concatenate([edge_index[1].astype(jnp.int32), ar % N])
    dst_deg = jnp.concatenate([edge_index[1].astype(jnp.int32),
                               N + (ar % CHUNK)])
    src_r = src.reshape(NW, C, CHUNK)
    dsta_r = dst_agg.reshape(NW, C, CHUNK)
    dstd_r = dst_deg.reshape(NW, C, CHUNK)

    zeros1 = jnp.zeros((RPS,), jnp.float32)
    zeros2 = jnp.zeros((CHUNK, D), jnp.float32)

    degp = _deg_kernel(dstd_r, zeros1)
    h1 = _tc0(x, W1)
    g1x, dinv = _tc1(h1, degp)

    s1 = _agg_kernel(g1x, src_r, dsta_r, zeros2)
    g2x = _tc2(s1, g1x, dinv, b1.reshape(1, D), g1.reshape(1, D),
               be1.reshape(1, D), W2)

    s2 = _agg_kernel(g2x, src_r, dsta_r, zeros2)
    out = _tc3(s2, g2x, dinv, b2.reshape(1, D), g2.reshape(1, D),
               be2.reshape(1, D), Wc, bc.reshape(1, 1))
    return out


# speedup vs baseline: 1.2625x; 1.0177x over previous
import functools

import jax
import jax.numpy as jnp
from jax import lax
from jax.experimental import pallas as pl
from jax.experimental.pallas import tpu as pltpu
from jax.experimental.pallas import tpu_sc as plsc

N = 10000
E = 320000
D = 128
EPS = 1e-5

NC = 2
NS = 16
NW = NC * NS
CHUNK = 128
NBUF = 2
IGRP = 40
C = 80
NGRP = C // IGRP
E_PAD = NW * C * CHUNK
ZPAD = 128
NROWS = 10240
RPS = NROWS // NS
NA = 10112
ARPS = NA // NS

_mesh = plsc.VectorSubcoreMesh(core_axis_name="c", subcore_axis_name="s")


@functools.partial(
    pl.kernel,
    out_type=jax.ShapeDtypeStruct((NC, NROWS), jnp.float32),
    mesh=_mesh,
    scratch_types=[
        pltpu.VMEM((C, CHUNK), jnp.int32),
        pltpu.VMEM((CHUNK,), jnp.float32),
        pltpu.VMEM_SHARED((NROWS,), jnp.float32),
    ],
)
def _deg_kernel(dst_hbm, zeros1_hbm, out_hbm, dst_v, ones_v, acc_sh):
    c = lax.axis_index("c")
    s = lax.axis_index("s")
    w = s * NC + c

    pltpu.sync_copy(dst_hbm.at[w], dst_v)
    for i in range(CHUNK // 16):
        ones_v[pl.ds(i * 16, 16)] = jnp.ones((16,), jnp.float32)

    pltpu.sync_copy(zeros1_hbm, acc_sh.at[pl.ds(s * RPS, RPS)])
    plsc.subcore_barrier()

    @pl.loop(0, C)
    def _(j):
        pltpu.sync_copy(ones_v, acc_sh.at[dst_v.at[j]], add=True)

    plsc.subcore_barrier()
    pltpu.sync_copy(acc_sh.at[pl.ds(s * RPS, RPS)],
                    out_hbm.at[c, pl.ds(s * RPS, RPS)])


@functools.partial(
    pl.kernel,
    out_type=jax.ShapeDtypeStruct((NC, NA, D), jnp.float32),
    mesh=_mesh,
    scratch_types=[
        pltpu.VMEM((C, CHUNK), jnp.int32),
        pltpu.VMEM((IGRP, CHUNK), jnp.int32),
        pltpu.VMEM((NBUF, CHUNK, D), jnp.float32),
        pltpu.VMEM_SHARED((NA, D), jnp.float32),
        [pltpu.SemaphoreType.DMA] * NBUF,
        [pltpu.SemaphoreType.DMA] * NBUF,
    ],
)
def _agg_kernel(g_hbm, src_hbm, dst_hbm, zeros2_hbm, out_hbm,
                src_v, dst_v, rows_v, acc_sh, gsems, ssems):
    c = lax.axis_index("c")
    s = lax.axis_index("s")
    w = s * NC + c

    pltpu.sync_copy(zeros2_hbm, rows_v.at[0])
    for i in range(ARPS // CHUNK):
        pltpu.async_copy(rows_v.at[0],
                         acc_sh.at[pl.ds(s * ARPS + i * CHUNK, CHUNK)],
                         gsems[0])
    rem = ARPS % CHUNK
    pltpu.sync_copy(zeros2_hbm.at[pl.ds(0, rem)],
                    acc_sh.at[pl.ds(s * ARPS + (ARPS // CHUNK) * CHUNK,
                                    rem)])
    for i in range(ARPS // CHUNK):
        pltpu.make_async_copy(rows_v.at[0],
                              acc_sh.at[pl.ds(s * ARPS + i * CHUNK,
                                              CHUNK)],
                              gsems[0]).wait()
    plsc.subcore_barrier()

    def start_gather(b, j):
        pltpu.async_copy(g_hbm.at[src_v.at[j]], rows_v.at[b], gsems[b])

    def wait_gather(b, j):
        pltpu.make_async_copy(g_hbm.at[src_v.at[j]], rows_v.at[b],
                              gsems[b]).wait()

    def start_scatter(b, j):
        pltpu.async_copy(rows_v.at[b], acc_sh.at[dst_v.at[j]], ssems[b],
                         add=True)

    def wait_scatter(b, j):
        pltpu.make_async_copy(rows_v.at[b], acc_sh.at[dst_v.at[j]],
                              ssems[b]).wait()

    pltpu.sync_copy(src_hbm.at[w], src_v)
    pltpu.sync_copy(dst_hbm.at[w, pl.ds(0, IGRP)], dst_v)

    for b in range(NBUF):
        start_gather(b, b)

    refill_t = IGRP // NBUF

    @pl.loop(0, C // NBUF - 1)
    def _(t):
        @pl.when(t == refill_t)
        def _():
            pltpu.sync_copy(dst_hbm.at[w, pl.ds(IGRP, IGRP)], dst_v)

        for b in range(NBUF):
            j = t * NBUF + b
            jm = lax.rem(j, IGRP)
            wait_gather(b, j)
            start_scatter(b, jm)
            wait_scatter(b, jm)
            start_gather(b, j + NBUF)

    for b in range(NBUF):
        j = C - NBUF + b
        wait_gather(b, j)
        start_scatter(b, lax.rem(j, IGRP))
        wait_scatter(b, lax.rem(j, IGRP))

    plsc.subcore_barrier()
    pltpu.sync_copy(acc_sh.at[pl.ds(s * ARPS, ARPS)],
                    out_hbm.at[c, pl.ds(s * ARPS, ARPS)])


def _tc0_body(x_ref, w_ref, h_ref):
    h_ref[...] = jnp.dot(x_ref[...], w_ref[...],
                         preferred_element_type=jnp.float32)


def _tc1_body(h_ref, degp_ref, g_ref, dinv_ref):
    deg = degp_ref[0, :N] + degp_ref[1, :N] + 1.0
    dinv = lax.rsqrt(deg)[:, None]
    g_ref[:N] = h_ref[...] * dinv
    g_ref[N:] = jnp.zeros((ZPAD, D), jnp.float32)
    dinv_ref[...] = dinv


def _tc2_body(s_ref, g_ref, dinv_ref, b_ref, gam_ref, bet_ref, w_ref,
              out_ref):
    agg = s_ref[0, :N] + s_ref[1, :N] + g_ref[:N]
    t = agg * dinv_ref[...] + b_ref[...]
    t = jnp.maximum(t, 0.0)
    mean = jnp.mean(t, axis=0, keepdims=True)
    var = jnp.mean(jnp.square(t - mean), axis=0, keepdims=True)
    bn = (t - mean) * lax.rsqrt(var + EPS) * gam_ref[...] + bet_ref[...]
    h2 = jnp.dot(bn, w_ref[...], preferred_element_type=jnp.float32)
    out_ref[:N] = h2 * dinv_ref[...]
    out_ref[N:] = jnp.zeros((ZPAD, D), jnp.float32)


def _tc3_body(s_ref, g_ref, dinv_ref, b_ref, gam_ref, bet_ref, wc_ref,
              bc_ref, out_ref):
    agg = s_ref[0, :N] + s_ref[1, :N] + g_ref[:N]
    t = agg * dinv_ref[...] + b_ref[...]
    t = jnp.maximum(t, 0.0)
    mean = jnp.mean(t, axis=0, keepdims=True)
    var = jnp.mean(jnp.square(t - mean), axis=0, keepdims=True)
    bn = (t - mean) * lax.rsqrt(var + EPS) * gam_ref[...] + bet_ref[...]
    out_ref[...] = (
        jnp.dot(bn, wc_ref[...], preferred_element_type=jnp.float32)
        + bc_ref[...])


_tc0 = pl.pallas_call(
    _tc0_body,
    out_shape=jax.ShapeDtypeStruct((N, D), jnp.float32),
)

_tc1 = pl.pallas_call(
    _tc1_body,
    out_shape=(jax.ShapeDtypeStruct((N + ZPAD, D), jnp.float32),
               jax.ShapeDtypeStruct((N, 1), jnp.float32)),
)

_tc2 = pl.pallas_call(
    _tc2_body,
    out_shape=jax.ShapeDtypeStruct((N + ZPAD, D), jnp.float32),
)

_tc3 = pl.pallas_call(
    _tc3_body,
    out_shape=jax.ShapeDtypeStruct((N, 1), jnp.float32),
)


def kernel(x, edge_index, W1, b1, g1, be1, W2, b2, g2, be2, Wc, bc):
    pad = E_PAD - E
    ar = jnp.arange(pad, dtype=jnp.int32)
    src = jnp.concatenate([edge_index[0].astype(jnp.int32),
                           N + (ar % ZPAD)])
    dst_agg = jnp.concatenate([edge_index[1].astype(jnp.int32), ar % N])
    dst_deg = jnp.concatenate([edge_index[1].astype(jnp.int32),
                               N + (ar % CHUNK)])
    src_r = src.reshape(NW, C, CHUNK)
    dsta_r = dst_agg.reshape(NW, C, CHUNK)
    dstd_r = dst_deg.reshape(NW, C, CHUNK)

    zeros1 = jnp.zeros((RPS,), jnp.float32)
    zeros2 = jnp.zeros((CHUNK, D), jnp.float32)

    degp = _deg_kernel(dstd_r, zeros1)
    h1 = _tc0(x, W1)
    g1x, dinv = _tc1(h1, degp)

    s1 = _agg_kernel(g1x, src_r, dsta_r, zeros2)
    g2x = _tc2(s1, g1x, dinv, b1.reshape(1, D), g1.reshape(1, D),
               be1.reshape(1, D), W2)

    s2 = _agg_kernel(g2x, src_r, dsta_r, zeros2)
    out = _tc3(s2, g2x, dinv, b2.reshape(1, D), g2.reshape(1, D),
               be2.reshape(1, D), Wc, bc.reshape(1, 1))
    return out
